# baseline (speedup 1.0000x reference)
import functools

import jax
import jax.numpy as jnp
from jax import lax
from jax.experimental import pallas as pl
from jax.experimental.pallas import tpu as pltpu
from jax.experimental.pallas import tpu_sc as plsc

NUM_EMB = 1000000
EMB_DIM = 64
BASE_DIM = 128
BATCH = 16384
FIELDS = 26

_NROWS = BATCH * FIELDS
_IDXW = 128
_IDX_ROWS = _NROWS // _IDXW

_NC, _NS = 2, 16
_NW = _NC * _NS

_IPW = _NROWS // _NW
_IDXR_PW = _IPW // _IDXW
_C = 256
_NSTREAM = _C // _IDXW
_STEPS = _IPW // _C

_PB = 16384
_SPLIT = 32 * _PB
_HI = NUM_EMB - _SPLIT
_SENT = -1


def _proj_body(xt_ref, w_ref, o_ref):
  o_ref[...] = lax.dot_general(
      xt_ref[...], w_ref[...],
      dimension_numbers=(((0,), (0,)), ((), ())),
      preferred_element_type=jnp.float32)


def _tc_project(tableT, Wt, nrows, blk0):
  grid = (nrows + _PB - 1) // _PB
  return pl.pallas_call(
      _proj_body,
      grid=(grid,),
      in_specs=[
          pl.BlockSpec((EMB_DIM, _PB), lambda i: (0, blk0 + i)),
          pl.BlockSpec((EMB_DIM, BASE_DIM), lambda i: (0, 0)),
      ],
      out_specs=pl.BlockSpec((_PB, BASE_DIM), lambda i: (i, 0)),
      out_shape=jax.ShapeDtypeStruct((nrows, BASE_DIM), jnp.float32),
  )(tableT, Wt)


_mesh = plsc.VectorSubcoreMesh(core_axis_name="c", subcore_axis_name="s")


def _pipeline(p_hbm, out_hbm, idx_v, oidx_v, g0, g1, sg0, sg1, sw0, sw1,
              row0, scatter):
  bufs = (g0, g1)
  gsems = (sg0, sg1)
  wsems = (sw0, sw1)

  def fire(t):
    buf, sem = bufs[t % 2], gsems[t % 2]
    return [
        pltpu.async_copy(
            p_hbm.at[plsc.Indices(idx_v.at[t * _NSTREAM + j],
                                  ignored_value=_SENT)],
            buf.at[pl.ds(j * _IDXW, _IDXW)],
            sem)
        for j in range(_NSTREAM)
    ]

  def write(t):
    buf, sem = bufs[t % 2], wsems[t % 2]
    if scatter:
      descs = []
      for j in range(_NSTREAM):
        descs.append(
            pltpu.async_copy(
                buf.at[pl.ds(j * _IDXW, _IDXW)],
                out_hbm.at[plsc.Indices(oidx_v.at[t * _NSTREAM + j],
                                        ignored_value=_SENT)],
                sem))
      return descs
    desc = pltpu.make_async_copy(
        buf, out_hbm.at[pl.ds(row0 + t * _C, _C)], sem)
    desc.start()
    return [desc]

  pend_g = fire(0)
  pend_w = [None, None]
  for t in range(_STEPS):
    b = t % 2
    if t + 1 < _STEPS:
      nb = (t + 1) % 2
      if pend_w[nb] is not None:
        for d in pend_w[nb]:
          d.wait()
        pend_w[nb] = None
      next_g = fire(t + 1)
    for cp in pend_g:
      cp.wait()
    if t + 1 < _STEPS:
      pend_g = next_g
    pend_w[b] = write(t)
  for ds_ in pend_w:
    if ds_ is not None:
      for d in ds_:
        d.wait()


def _sc_gather_a(proj, gidx2d):
  @functools.partial(
      pl.kernel,
      mesh=_mesh,
      out_type=jax.ShapeDtypeStruct((_NROWS, BASE_DIM), jnp.float32),
      scratch_types=[
          pltpu.VMEM((_IDXR_PW, _IDXW), jnp.int32),
          pltpu.VMEM((_C, BASE_DIM), jnp.float32),
          pltpu.VMEM((_C, BASE_DIM), jnp.float32),
          pltpu.SemaphoreType.DMA,
          pltpu.SemaphoreType.DMA,
          pltpu.SemaphoreType.DMA,
          pltpu.SemaphoreType.DMA,
      ],
  )
  def k(p_hbm, idx_hbm, out_hbm, idx_v, g0, g1, sg0, sg1, sw0, sw1):
    wid = lax.axis_index("s") * _NC + lax.axis_index("c")
    pltpu.sync_copy(idx_hbm.at[pl.ds(wid * _IDXR_PW, _IDXR_PW)], idx_v)
    _pipeline(p_hbm, out_hbm, idx_v, None, g0, g1, sg0, sg1, sw0, sw1,
              wid * _IPW, scatter=False)

  return k(proj, gidx2d)


def _sc_gather_b(proj, gidx2d, oidx2d, out):
  @functools.partial(
      pl.kernel,
      mesh=_mesh,
      out_type=(),
      scratch_types=[
          pltpu.VMEM((_IDXR_PW, _IDXW), jnp.int32),
          pltpu.VMEM((_IDXR_PW, _IDXW), jnp.int32),
          pltpu.VMEM((_C, BASE_DIM), jnp.float32),
          pltpu.VMEM((_C, BASE_DIM), jnp.float32),
          pltpu.SemaphoreType.DMA,
          pltpu.SemaphoreType.DMA,
          pltpu.SemaphoreType.DMA,
          pltpu.SemaphoreType.DMA,
      ],
  )
  def k(p_hbm, idx_hbm, oidx_hbm, out_hbm,
        idx_v, oidx_v, g0, g1, sg0, sg1, sw0, sw1):
    wid = lax.axis_index("s") * _NC + lax.axis_index("c")
    pltpu.sync_copy(idx_hbm.at[pl.ds(wid * _IDXR_PW, _IDXR_PW)], idx_v)
    pltpu.sync_copy(oidx_hbm.at[pl.ds(wid * _IDXR_PW, _IDXR_PW)], oidx_v)
    _pipeline(p_hbm, out_hbm, idx_v, oidx_v, g0, g1, sg0, sg1, sw0, sw1,
              wid * _IPW, scatter=True)

  out_ref = jax.new_ref(out)
  k(proj, gidx2d, oidx2d, out_ref)
  return out_ref[...]


def kernel(input, table, W):
  idxf = input.astype(jnp.int32).T.reshape(_NROWS)
  in_low = idxf < _SPLIT
  gidx_a = jnp.where(in_low, idxf, _SENT).reshape(_IDX_ROWS, _IDXW)
  gidx_b = jnp.where(in_low, _SENT, idxf - _SPLIT).reshape(_IDX_ROWS, _IDXW)
  rowpos = jnp.arange(_NROWS, dtype=jnp.int32)
  oidx_b = jnp.where(in_low, _SENT, rowpos).reshape(_IDX_ROWS, _IDXW)

  tableT = table.T
  Wt = W.T
  p_low = _tc_project(tableT, Wt, _SPLIT, 0)
  out_a = _sc_gather_a(p_low, gidx_a)
  p_high = _tc_project(tableT, Wt, _HI, _SPLIT // _PB)
  rows = _sc_gather_b(p_high, gidx_b, oidx_b, out_a)
  return rows.reshape(FIELDS, BATCH, BASE_DIM).transpose(1, 0, 2)


# speedup vs baseline: 1.1567x; 1.1567x over previous
import functools

import jax
import jax.numpy as jnp
from jax import lax
from jax.experimental import pallas as pl
from jax.experimental.pallas import tpu as pltpu
from jax.experimental.pallas import tpu_sc as plsc

NUM_EMB = 1000000
EMB_DIM = 64
BASE_DIM = 128
BATCH = 16384
FIELDS = 26

_NROWS = BATCH * FIELDS
_IDXW = 128
_IDX_ROWS = _NROWS // _IDXW

_NC, _NS = 2, 16
_NW = _NC * _NS

_IPW = _NROWS // _NW
_IDXR_PW = _IPW // _IDXW
_C = 256
_NSTREAM = _C // _IDXW
_STEPS = _IPW // _C

_PB = 16384
_PGRID = (NUM_EMB + _PB - 1) // _PB


def _proj_body(xt_ref, w_ref, o_ref):
  o_ref[...] = lax.dot_general(
      xt_ref[...], w_ref[...],
      dimension_numbers=(((0,), (0,)), ((), ())),
      preferred_element_type=jnp.float32)


def _tc_project_table(tableT, Wt):
  return pl.pallas_call(
      _proj_body,
      grid=(_PGRID,),
      in_specs=[
          pl.BlockSpec((EMB_DIM, _PB), lambda i: (0, i)),
          pl.BlockSpec((EMB_DIM, BASE_DIM), lambda i: (0, 0)),
      ],
      out_specs=pl.BlockSpec((_PB, BASE_DIM), lambda i: (i, 0)),
      out_shape=jax.ShapeDtypeStruct((NUM_EMB, BASE_DIM), jnp.float32),
  )(tableT, Wt)


def _sc_gather(proj, idx2d):
  mesh = plsc.VectorSubcoreMesh(core_axis_name="c", subcore_axis_name="s")

  @functools.partial(
      pl.kernel,
      mesh=mesh,
      out_type=jax.ShapeDtypeStruct((_NROWS, BASE_DIM), jnp.float32),
      scratch_types=[
          pltpu.VMEM((_IDXR_PW, _IDXW), jnp.int32),
          pltpu.VMEM((_C, BASE_DIM), jnp.float32),
          pltpu.VMEM((_C, BASE_DIM), jnp.float32),
          pltpu.SemaphoreType.DMA,
          pltpu.SemaphoreType.DMA,
          pltpu.SemaphoreType.DMA,
          pltpu.SemaphoreType.DMA,
      ],
  )
  def k(p_hbm, idx_hbm, out_hbm, idx_v, g0, g1, sg0, sg1, sw0, sw1):
    wid = lax.axis_index("s") * _NC + lax.axis_index("c")
    row0 = wid * _IPW

    pltpu.sync_copy(idx_hbm.at[pl.ds(wid * _IDXR_PW, _IDXR_PW)], idx_v)

    bufs = (g0, g1)
    gsems = (sg0, sg1)
    wsems = (sw0, sw1)

    def fire(t):
      buf, sem = bufs[t % 2], gsems[t % 2]
      return [
          pltpu.async_copy(
              p_hbm.at[idx_v.at[t * _NSTREAM + j]],
              buf.at[pl.ds(j * _IDXW, _IDXW)],
              sem)
          for j in range(_NSTREAM)
      ]

    pend_g = fire(0)
    pend_w = [None, None]
    for t in range(_STEPS):
      b = t % 2
      if t + 1 < _STEPS:
        nb = (t + 1) % 2
        if pend_w[nb] is not None:
          pend_w[nb].wait()
          pend_w[nb] = None
        next_g = fire(t + 1)
      for cp in pend_g:
        cp.wait()
      if t + 1 < _STEPS:
        pend_g = next_g
      desc = pltpu.make_async_copy(
          bufs[b], out_hbm.at[pl.ds(row0 + t * _C, _C)], wsems[b])
      desc.start()
      pend_w[b] = desc
    for d in pend_w:
      if d is not None:
        d.wait()

  return k(proj, idx2d)


def kernel(input, table, W):
  idx2d = input.astype(jnp.int32).T.reshape(_IDX_ROWS, _IDXW)
  proj = _tc_project_table(table.T, W.T)
  rows = _sc_gather(proj, idx2d)
  return rows.reshape(FIELDS, BATCH, BASE_DIM).transpose(1, 0, 2)


# speedup vs baseline: 1.1727x; 1.0138x over previous
import functools

import jax
import jax.numpy as jnp
from jax import lax
from jax.experimental import pallas as pl
from jax.experimental.pallas import tpu as pltpu
from jax.experimental.pallas import tpu_sc as plsc

NUM_EMB = 1000000
EMB_DIM = 64
BASE_DIM = 128
BATCH = 16384
FIELDS = 26

_NROWS = BATCH * FIELDS
_IDXW = 128
_IDX_ROWS = _NROWS // _IDXW

_NC, _NS = 2, 16
_NW = _NC * _NS

_IPW = _NROWS // _NW
_IDXR_PW = _IPW // _IDXW
_C = 256
_NSTREAM = _C // _IDXW
_STEPS = _IPW // _C

_PB = 32768
_PGRID = (NUM_EMB + _PB - 1) // _PB


def _proj_body(xt_ref, w_ref, o_ref):
  o_ref[...] = lax.dot_general(
      xt_ref[...], w_ref[...],
      dimension_numbers=(((0,), (0,)), ((), ())),
      preferred_element_type=jnp.float32)


def _tc_project_table(tableT, Wt):
  return pl.pallas_call(
      _proj_body,
      grid=(_PGRID,),
      in_specs=[
          pl.BlockSpec((EMB_DIM, _PB), lambda i: (0, i)),
          pl.BlockSpec((EMB_DIM, BASE_DIM), lambda i: (0, 0)),
      ],
      out_specs=pl.BlockSpec((_PB, BASE_DIM), lambda i: (i, 0)),
      out_shape=jax.ShapeDtypeStruct((NUM_EMB, BASE_DIM), jnp.float32),
  )(tableT, Wt)


def _sc_gather(proj, idx2d):
  mesh = plsc.VectorSubcoreMesh(core_axis_name="c", subcore_axis_name="s")

  @functools.partial(
      pl.kernel,
      mesh=mesh,
      out_type=jax.ShapeDtypeStruct((_NROWS, BASE_DIM), jnp.float32),
      scratch_types=[
          pltpu.VMEM((_IDXR_PW, _IDXW), jnp.int32),
          pltpu.VMEM((_C, BASE_DIM), jnp.float32),
          pltpu.VMEM((_C, BASE_DIM), jnp.float32),
          pltpu.SemaphoreType.DMA,
          pltpu.SemaphoreType.DMA,
          pltpu.SemaphoreType.DMA,
          pltpu.SemaphoreType.DMA,
      ],
  )
  def k(p_hbm, idx_hbm, out_hbm, idx_v, g0, g1, sg0, sg1, sw0, sw1):
    wid = lax.axis_index("s") * _NC + lax.axis_index("c")
    row0 = wid * _IPW

    pltpu.sync_copy(idx_hbm.at[pl.ds(wid * _IDXR_PW, _IDXR_PW)], idx_v)

    bufs = (g0, g1)
    gsems = (sg0, sg1)
    wsems = (sw0, sw1)

    def fire(t):
      buf, sem = bufs[t % 2], gsems[t % 2]
      return [
          pltpu.async_copy(
              p_hbm.at[idx_v.at[t * _NSTREAM + j]],
              buf.at[pl.ds(j * _IDXW, _IDXW)],
              sem)
          for j in range(_NSTREAM)
      ]

    pend_g = fire(0)
    pend_w = [None, None]
    for t in range(_STEPS):
      b = t % 2
      if t + 1 < _STEPS:
        nb = (t + 1) % 2
        if pend_w[nb] is not None:
          pend_w[nb].wait()
          pend_w[nb] = None
        next_g = fire(t + 1)
      for cp in pend_g:
        cp.wait()
      if t + 1 < _STEPS:
        pend_g = next_g
      desc = pltpu.make_async_copy(
          bufs[b], out_hbm.at[pl.ds(row0 + t * _C, _C)], wsems[b])
      desc.start()
      pend_w[b] = desc
    for d in pend_w:
      if d is not None:
        d.wait()

  return k(proj, idx2d)


def kernel(input, table, W):
  idx2d = input.astype(jnp.int32).T.reshape(_IDX_ROWS, _IDXW)
  proj = _tc_project_table(table.T, W.T)
  rows = _sc_gather(proj, idx2d)
  return rows.reshape(FIELDS, BATCH, BASE_DIM).transpose(1, 0, 2)
